# baseline (speedup 1.0000x reference)
import functools

import jax
import jax.numpy as jnp
from jax import lax
from jax.experimental import pallas as pl
from jax.experimental.pallas import tpu as pltpu
from jax.experimental.pallas import tpu_sc as plsc

N = 10000
NP = 10240
D = 128
E = 320000
HID = 256
C = 6

NW = 32
EPW = E // NW
K = 64
NCH = 157
EPWP = NCH * K
TRASH = N + 100
NPU = 10112
SROW = 10016
BM = 1024
NR = NP // D
RPT = NR // 16


def _first_body(x_ref, w_ref, as_ref, ad_ref, h_ref, s_ref, d_ref):
    h = jnp.dot(x_ref[...], w_ref[...], preferred_element_type=jnp.float32)
    h_ref[...] = h
    s_ref[...] = jnp.dot(h, as_ref[...],
                         preferred_element_type=jnp.float32).reshape(
                             BM // D, D)
    d_ref[...] = jnp.dot(h, ad_ref[...],
                         preferred_element_type=jnp.float32).reshape(
                             BM // D, D)


def _tc_first(xp, w, a_s, a_d):
    return pl.pallas_call(
        _first_body,
        grid=(NP // BM,),
        in_specs=[
            pl.BlockSpec((BM, D), lambda i: (i, 0)),
            pl.BlockSpec((D, D), lambda i: (0, 0)),
            pl.BlockSpec((D,), lambda i: (0,)),
            pl.BlockSpec((D,), lambda i: (0,)),
        ],
        out_specs=[
            pl.BlockSpec((BM, D), lambda i: (i, 0)),
            pl.BlockSpec((BM // D, D), lambda i: (i, 0)),
            pl.BlockSpec((BM // D, D), lambda i: (i, 0)),
        ],
        out_shape=[
            jax.ShapeDtypeStruct((NP, D), jnp.float32),
            jax.ShapeDtypeStruct((NR, D), jnp.float32),
            jax.ShapeDtypeStruct((NR, D), jnp.float32),
        ],
    )(xp, w, a_s, a_d)


def _combine(u_ref, sd_ref, b_ref):
    u = u_ref[0] + u_ref[1]
    s = sd_ref[0] + sd_ref[1]
    recip = 1.0 / jnp.where(s == 0.0, 1.0, s)
    eye = (lax.broadcasted_iota(jnp.int32, (D, D), 0)
           == lax.broadcasted_iota(jnp.int32, (D, D), 1))
    rows = []
    for g in range(BM // D):
        dg = jnp.where(eye, recip[g][None, :], 0.0)
        rows.append(jnp.dot(dg, u[g * D:(g + 1) * D],
                            preferred_element_type=jnp.float32))
    return jnp.maximum(jnp.concatenate(rows, axis=0) + b_ref[...], 0.0)


def _mid_body(u_ref, sd_ref, b_ref, w_ref, as_ref, ad_ref,
              h_ref, s_ref, d_ref):
    o = _combine(u_ref, sd_ref, b_ref)
    h = jnp.dot(o, w_ref[...], preferred_element_type=jnp.float32)
    h_ref[...] = h
    s_ref[...] = jnp.dot(h, as_ref[...],
                         preferred_element_type=jnp.float32).reshape(
                             BM // D, D)
    d_ref[...] = jnp.dot(h, ad_ref[...],
                         preferred_element_type=jnp.float32).reshape(
                             BM // D, D)


def _tc_mid(u, sd, b, w, a_s, a_d):
    return pl.pallas_call(
        _mid_body,
        grid=(NP // BM,),
        in_specs=[
            pl.BlockSpec((2, BM, D), lambda i: (0, i, 0)),
            pl.BlockSpec((2, BM // D, D), lambda i: (0, i, 0)),
            pl.BlockSpec((D,), lambda i: (0,)),
            pl.BlockSpec((D, D), lambda i: (0, 0)),
            pl.BlockSpec((D,), lambda i: (0,)),
            pl.BlockSpec((D,), lambda i: (0,)),
        ],
        out_specs=[
            pl.BlockSpec((BM, D), lambda i: (i, 0)),
            pl.BlockSpec((BM // D, D), lambda i: (i, 0)),
            pl.BlockSpec((BM // D, D), lambda i: (i, 0)),
        ],
        out_shape=[
            jax.ShapeDtypeStruct((NP, D), jnp.float32),
            jax.ShapeDtypeStruct((NR, D), jnp.float32),
            jax.ShapeDtypeStruct((NR, D), jnp.float32),
        ],
    )(u, sd, b, w, a_s, a_d)


def _head_body(u_ref, sd_ref, b_ref, w1_ref, b1_ref, w2_ref, b2_ref, y_ref):
    o = _combine(u_ref, sd_ref, b_ref)
    t = jnp.maximum(
        jnp.dot(o, w1_ref[...], preferred_element_type=jnp.float32)
        + b1_ref[...], 0.0)
    y_ref[...] = (jnp.dot(t, w2_ref[...], preferred_element_type=jnp.float32)
                  + b2_ref[...])


def _tc_head(u, sd, b, w1, b1, w2p, b2p):
    return pl.pallas_call(
        _head_body,
        grid=(NP // BM,),
        in_specs=[
            pl.BlockSpec((2, BM, D), lambda i: (0, i, 0)),
            pl.BlockSpec((2, BM // D, D), lambda i: (0, i, 0)),
            pl.BlockSpec((D,), lambda i: (0,)),
            pl.BlockSpec((D, HID), lambda i: (0, 0)),
            pl.BlockSpec((HID,), lambda i: (0,)),
            pl.BlockSpec((HID, D), lambda i: (0, 0)),
            pl.BlockSpec((D,), lambda i: (0,)),
        ],
        out_specs=[pl.BlockSpec((BM, D), lambda i: (i, 0))],
        out_shape=[jax.ShapeDtypeStruct((NP, D), jnp.float32)],
    )(u, sd, b, w1, b1, w2p, b2p)[0]


def _sc_ee_body(asrc_hbm, adst_hbm, esrc_hbm, edst_hbm,
                ee_hbm, sd_hbm,
                src1d, dst1d, ee_all, asrc_v, adst_v, s_local, iota_v,
                s_sh):
    cid = lax.axis_index("c")
    sid = lax.axis_index("s")
    wid = sid * 2 + cid
    base = wid * EPWP

    zvec = jnp.zeros((16,), jnp.float32)
    iota16 = lax.iota(jnp.int32, 16)

    def _zs_body(r, _):
        for j in range(D // 16):
            s_local[r, pl.ds(16 * j, 16)] = zvec
        return 0
    lax.fori_loop(0, NR, _zs_body, 0)

    @pl.when(sid < NR // 8)
    def _():
        pltpu.sync_copy(s_local.at[pl.ds(0, 8)], s_sh.at[pl.ds(sid * 8, 8)])
    plsc.subcore_barrier()

    for i in range(NR // 16):
        iota_v[pl.ds(16 * i, 16)] = iota16 + 16 * i
    pltpu.sync_copy(asrc_hbm, asrc_v)
    pltpu.sync_copy(adst_hbm, adst_v)
    pltpu.sync_copy(esrc_hbm.at[pl.ds(base, EPWP)], src1d)
    pltpu.sync_copy(edst_hbm.at[pl.ds(base, EPWP)], dst1d)

    @plsc.parallel_loop(0, EPWP // 16, unroll=8)
    def _ee_body(i):
        sv = src1d[pl.ds(16 * i, 16)]
        dv = dst1d[pl.ds(16 * i, 16)]
        av = plsc.load_gather(asrc_v, [sv >> 7, sv & 127])
        bv = plsc.load_gather(adst_v, [dv >> 7, dv & 127])
        e = av + bv
        e = jnp.where(e >= 0.0, e, e * 0.2)
        ee = jnp.exp(e)
        lim = jnp.full((16,), EPW - 16 * i, jnp.int32)
        ee_all[pl.ds(16 * i, 16)] = jnp.where(iota16 < lim, ee, 0.0)

    def _s_body(i, _):
        dv = dst1d[pl.ds(16 * i, 16)]
        ee = ee_all[pl.ds(16 * i, 16)]
        plsc.addupdate_scatter(s_local, [dv >> 7, dv & 127], ee)
        return 0
    lax.fori_loop(0, EPWP // 16, _s_body, 0)

    pltpu.sync_copy(s_local, s_sh.at[iota_v], add=True)
    pltpu.sync_copy(ee_all, ee_hbm.at[pl.ds(base, EPWP)])

    plsc.subcore_barrier()

    @pl.when(sid < NR // 8)
    def _():
        pltpu.sync_copy(s_sh.at[pl.ds(sid * 8, 8)],
                        sd_hbm.at[cid, pl.ds(sid * 8, 8)])


_sc_ee = functools.partial(
    pl.kernel,
    out_type=(
        jax.ShapeDtypeStruct((NW * EPWP,), jnp.float32),
        jax.ShapeDtypeStruct((2, NR, D), jnp.float32),
    ),
    mesh=plsc.VectorSubcoreMesh(core_axis_name="c", subcore_axis_name="s"),
    compiler_params=pltpu.CompilerParams(needs_layout_passes=False),
    scratch_types=[
        pltpu.VMEM((EPWP,), jnp.int32),
        pltpu.VMEM((EPWP,), jnp.int32),
        pltpu.VMEM((EPWP,), jnp.float32),
        pltpu.VMEM((NR, D), jnp.float32),
        pltpu.VMEM((NR, D), jnp.float32),
        pltpu.VMEM((NR, D), jnp.float32),
        pltpu.VMEM((NR,), jnp.int32),
        pltpu.VMEM_SHARED((NR, D), jnp.float32),
    ],
)(_sc_ee_body)


def _sc_agg_body(h_hbm, esrc_hbm, edst_hbm, ee_hbm, u_hbm,
                 rsrc, rdst, ree, gbuf, sbuf, gsem, ssem, isem, u_sh):
    cid = lax.axis_index("c")
    sid = lax.axis_index("s")
    wid = sid * 2 + cid
    base = wid * EPWP

    zvec = jnp.zeros((16,), jnp.float32)

    def _zero_body(r, _):
        for j in range(D // 16):
            sbuf[0, r, pl.ds(16 * j, 16)] = zvec
        return 0
    lax.fori_loop(0, K, _zero_body, 0)
    for kk in range(NPU // 16 // K):
        pltpu.sync_copy(sbuf.at[0],
                        u_sh.at[pl.ds(sid * (NPU // 16) + kk * K, K)])
    pltpu.sync_copy(sbuf.at[0, pl.ds(0, (NPU // 16) % K)],
                    u_sh.at[pl.ds(sid * (NPU // 16) + (NPU // 16 // K) * K,
                                  (NPU // 16) % K)])
    plsc.subcore_barrier()

    def _issue_idx(c):
        slot = c % 6
        pltpu.async_copy(esrc_hbm.at[pl.ds(base + K * c, K)],
                         rsrc.at[slot], isem.at[slot])
        pltpu.async_copy(edst_hbm.at[pl.ds(base + K * c, K)],
                         rdst.at[slot], isem.at[slot])
        pltpu.async_copy(ee_hbm.at[pl.ds(base + K * c, K)],
                         ree.at[slot], isem.at[slot])

    def _wait_idx(c):
        slot = c % 6
        pltpu.make_async_copy(esrc_hbm.at[pl.ds(base + K * c, K)],
                              rsrc.at[slot], isem.at[slot]).wait()
        pltpu.make_async_copy(edst_hbm.at[pl.ds(base + K * c, K)],
                              rdst.at[slot], isem.at[slot]).wait()
        pltpu.make_async_copy(ee_hbm.at[pl.ds(base + K * c, K)],
                              ree.at[slot], isem.at[slot]).wait()

    for c in range(5):
        _issue_idx(c)
    for c in range(3):
        _wait_idx(c)
        pltpu.async_copy(h_hbm.at[rsrc.at[c]], gbuf.at[c], gsem.at[c])

    def _chunk_body(c, _):
        b = c % 3
        sb = c % 2
        slot = c % 6
        pltpu.make_async_copy(h_hbm.at[rsrc.at[slot]], gbuf.at[b],
                              gsem.at[b]).wait()

        @pl.when(c >= 2)
        def _():
            pltpu.make_async_copy(sbuf.at[sb], u_sh.at[rdst.at[slot]],
                                  ssem.at[sb]).wait()

        @pl.when(c + 5 < NCH)
        def _():
            _issue_idx(c + 5)

        @plsc.parallel_loop(0, K, unroll=8)
        def _scale_body(e):
            eev = plsc.load_gather(
                ree, [jnp.full((16,), slot, jnp.int32),
                      jnp.full((16,), e, jnp.int32)])
            for j in range(D // 16):
                sbuf[sb, e, pl.ds(16 * j, 16)] = (
                    gbuf[b, e, pl.ds(16 * j, 16)] * eev)

        @pl.when(c + 3 < NCH)
        def _():
            _wait_idx(c + 3)
            pltpu.async_copy(h_hbm.at[rsrc.at[(c + 3) % 6]], gbuf.at[b],
                             gsem.at[b])

        pltpu.async_copy(sbuf.at[sb], u_sh.at[rdst.at[slot]], ssem.at[sb],
                         add=True)
        return 0
    lax.fori_loop(0, NCH, _chunk_body, 0)

    for t in (NCH - 2, NCH - 1):
        pltpu.make_async_copy(sbuf.at[t % 2], u_sh.at[rdst.at[t % 6]],
                              ssem.at[t % 2]).wait()

    plsc.subcore_barrier()
    pltpu.sync_copy(u_sh.at[pl.ds(sid * (NPU // 16), NPU // 16)],
                    u_hbm.at[cid, pl.ds(sid * (NPU // 16), NPU // 16)])


_sc_agg = functools.partial(
    pl.kernel,
    out_type=jax.ShapeDtypeStruct((2, NP, D), jnp.float32),
    mesh=plsc.VectorSubcoreMesh(core_axis_name="c", subcore_axis_name="s"),
    compiler_params=pltpu.CompilerParams(needs_layout_passes=False),
    scratch_types=[
        pltpu.VMEM((6, K), jnp.int32),
        pltpu.VMEM((6, K), jnp.int32),
        pltpu.VMEM((6, K), jnp.float32),
        pltpu.VMEM((3, K, D), jnp.float32),
        pltpu.VMEM((2, K, D), jnp.float32),
        pltpu.SemaphoreType.DMA((3,)),
        pltpu.SemaphoreType.DMA((2,)),
        pltpu.SemaphoreType.DMA((6,)),
        pltpu.VMEM_SHARED((NPU, D), jnp.float32),
    ],
)(_sc_agg_body)


def kernel(x, edge_index, edge_weight, W0, a0s, a0d, b0, W1, a1s, a1d, b1,
           W2, a2s, a2d, b2, Wm1, bm1, Wm2, bm2):
    del edge_weight
    xp = jnp.zeros((NP, D), jnp.float32).at[:N].set(x)
    eidx = edge_index.astype(jnp.int32)
    w2p = jnp.zeros((HID, D), jnp.float32).at[:, :C].set(Wm2)
    b2p = jnp.zeros((D,), jnp.float32).at[:C].set(bm2)
    esrc = jnp.zeros((NW, EPWP), jnp.int32).at[:, :EPW].set(
        eidx[0].reshape(NW, EPW)).reshape(-1)
    edst = jnp.full((NW, EPWP), TRASH, jnp.int32).at[:, :EPW].set(
        eidx[1].reshape(NW, EPW)).reshape(-1)

    h, a_s, a_d = _tc_first(xp, W0, a0s, a0d)
    ee, sd = _sc_ee(a_s, a_d, esrc, edst)
    u = _sc_agg(h, esrc, edst, ee)
    h, a_s, a_d = _tc_mid(u, sd, b0, W1, a1s, a1d)
    ee, sd = _sc_ee(a_s, a_d, esrc, edst)
    u = _sc_agg(h, esrc, edst, ee)
    h, a_s, a_d = _tc_mid(u, sd, b1, W2, a2s, a2d)
    ee, sd = _sc_ee(a_s, a_d, esrc, edst)
    u = _sc_agg(h, esrc, edst, ee)
    y = _tc_head(u, sd, b2, Wm1, bm1, w2p, b2p)
    return y[:N, :C]


# speedup vs baseline: 1.0094x; 1.0094x over previous
import functools

import jax
import jax.numpy as jnp
from jax import lax
from jax.experimental import pallas as pl
from jax.experimental.pallas import tpu as pltpu
from jax.experimental.pallas import tpu_sc as plsc

N = 10000
NP = 10240
D = 128
E = 320000
HID = 256
C = 6

NW = 32
EPW = E // NW
K = 32
NCH = 313
EPWP = NCH * K
TRASH = N + 100
NPU = 10112
SROW = 10016
BM = 1024
NR = NP // D
RPT = NR // 16


def _first_body(x_ref, w_ref, as_ref, ad_ref, h_ref, s_ref, d_ref):
    h = jnp.dot(x_ref[...], w_ref[...], preferred_element_type=jnp.float32)
    h_ref[...] = h
    s_ref[...] = jnp.dot(h, as_ref[...],
                         preferred_element_type=jnp.float32).reshape(
                             BM // D, D)
    d_ref[...] = jnp.dot(h, ad_ref[...],
                         preferred_element_type=jnp.float32).reshape(
                             BM // D, D)


def _tc_first(xp, w, a_s, a_d):
    return pl.pallas_call(
        _first_body,
        grid=(NP // BM,),
        in_specs=[
            pl.BlockSpec((BM, D), lambda i: (i, 0)),
            pl.BlockSpec((D, D), lambda i: (0, 0)),
            pl.BlockSpec((D,), lambda i: (0,)),
            pl.BlockSpec((D,), lambda i: (0,)),
        ],
        out_specs=[
            pl.BlockSpec((BM, D), lambda i: (i, 0)),
            pl.BlockSpec((BM // D, D), lambda i: (i, 0)),
            pl.BlockSpec((BM // D, D), lambda i: (i, 0)),
        ],
        out_shape=[
            jax.ShapeDtypeStruct((NP, D), jnp.float32),
            jax.ShapeDtypeStruct((NR, D), jnp.float32),
            jax.ShapeDtypeStruct((NR, D), jnp.float32),
        ],
    )(xp, w, a_s, a_d)


def _combine(u_ref, sd_ref, b_ref):
    u = u_ref[0] + u_ref[1]
    s = sd_ref[0] + sd_ref[1]
    recip = 1.0 / jnp.where(s == 0.0, 1.0, s)
    eye = (lax.broadcasted_iota(jnp.int32, (D, D), 0)
           == lax.broadcasted_iota(jnp.int32, (D, D), 1))
    rows = []
    for g in range(BM // D):
        dg = jnp.where(eye, recip[g][None, :], 0.0)
        rows.append(jnp.dot(dg, u[g * D:(g + 1) * D],
                            preferred_element_type=jnp.float32))
    return jnp.maximum(jnp.concatenate(rows, axis=0) + b_ref[...], 0.0)


def _mid_body(u_ref, sd_ref, b_ref, w_ref, as_ref, ad_ref,
              h_ref, s_ref, d_ref):
    o = _combine(u_ref, sd_ref, b_ref)
    h = jnp.dot(o, w_ref[...], preferred_element_type=jnp.float32)
    h_ref[...] = h
    s_ref[...] = jnp.dot(h, as_ref[...],
                         preferred_element_type=jnp.float32).reshape(
                             BM // D, D)
    d_ref[...] = jnp.dot(h, ad_ref[...],
                         preferred_element_type=jnp.float32).reshape(
                             BM // D, D)


def _tc_mid(u, sd, b, w, a_s, a_d):
    return pl.pallas_call(
        _mid_body,
        grid=(NP // BM,),
        in_specs=[
            pl.BlockSpec((2, BM, D), lambda i: (0, i, 0)),
            pl.BlockSpec((2, BM // D, D), lambda i: (0, i, 0)),
            pl.BlockSpec((D,), lambda i: (0,)),
            pl.BlockSpec((D, D), lambda i: (0, 0)),
            pl.BlockSpec((D,), lambda i: (0,)),
            pl.BlockSpec((D,), lambda i: (0,)),
        ],
        out_specs=[
            pl.BlockSpec((BM, D), lambda i: (i, 0)),
            pl.BlockSpec((BM // D, D), lambda i: (i, 0)),
            pl.BlockSpec((BM // D, D), lambda i: (i, 0)),
        ],
        out_shape=[
            jax.ShapeDtypeStruct((NP, D), jnp.float32),
            jax.ShapeDtypeStruct((NR, D), jnp.float32),
            jax.ShapeDtypeStruct((NR, D), jnp.float32),
        ],
    )(u, sd, b, w, a_s, a_d)


def _head_body(u_ref, sd_ref, b_ref, w1_ref, b1_ref, w2_ref, b2_ref, y_ref):
    o = _combine(u_ref, sd_ref, b_ref)
    t = jnp.maximum(
        jnp.dot(o, w1_ref[...], preferred_element_type=jnp.float32)
        + b1_ref[...], 0.0)
    y_ref[...] = (jnp.dot(t, w2_ref[...], preferred_element_type=jnp.float32)
                  + b2_ref[...])


def _tc_head(u, sd, b, w1, b1, w2p, b2p):
    return pl.pallas_call(
        _head_body,
        grid=(NP // BM,),
        in_specs=[
            pl.BlockSpec((2, BM, D), lambda i: (0, i, 0)),
            pl.BlockSpec((2, BM // D, D), lambda i: (0, i, 0)),
            pl.BlockSpec((D,), lambda i: (0,)),
            pl.BlockSpec((D, HID), lambda i: (0, 0)),
            pl.BlockSpec((HID,), lambda i: (0,)),
            pl.BlockSpec((HID, D), lambda i: (0, 0)),
            pl.BlockSpec((D,), lambda i: (0,)),
        ],
        out_specs=[pl.BlockSpec((BM, D), lambda i: (i, 0))],
        out_shape=[jax.ShapeDtypeStruct((NP, D), jnp.float32)],
    )(u, sd, b, w1, b1, w2p, b2p)[0]


def _sc_body(h_hbm, asrc_hbm, adst_hbm, esrc_hbm, edst_hbm,
             u_hbm, sd_hbm,
             rsrc, rdst, ree, asrc_v, adst_v, s_local, iota_v,
             gbuf, sbuf, gsem, ssem, isem, u_sh):
    cid = lax.axis_index("c")
    sid = lax.axis_index("s")
    wid = sid * 2 + cid
    base = wid * EPWP

    zvec = jnp.zeros((16,), jnp.float32)
    iota16 = lax.iota(jnp.int32, 16)

    def _zero_body(r, _):
        for j in range(D // 16):
            sbuf[0, r, pl.ds(16 * j, 16)] = zvec
        return 0
    lax.fori_loop(0, K, _zero_body, 0)
    for kk in range(NPU // 16 // K):
        pltpu.sync_copy(sbuf.at[0],
                        u_sh.at[pl.ds(sid * (NPU // 16) + kk * K, K)])
    pltpu.sync_copy(sbuf.at[0, pl.ds(0, (NPU // 16) % K)],
                    u_sh.at[pl.ds(sid * (NPU // 16) + (NPU // 16 // K) * K,
                                  (NPU // 16) % K)])
    plsc.subcore_barrier()

    def _zs_body(r, _):
        for j in range(D // 16):
            s_local[r, pl.ds(16 * j, 16)] = zvec
        return 0
    lax.fori_loop(0, NR, _zs_body, 0)
    for i in range(NR // 16):
        iota_v[pl.ds(16 * i, 16)] = iota16 + (16 * i + SROW)
    pltpu.sync_copy(asrc_hbm, asrc_v)
    pltpu.sync_copy(adst_hbm, adst_v)

    def _issue_idx(c):
        slot = c % 6
        pltpu.async_copy(esrc_hbm.at[pl.ds(base + K * c, K)],
                         rsrc.at[slot], isem.at[slot])
        pltpu.async_copy(edst_hbm.at[pl.ds(base + K * c, K)],
                         rdst.at[slot], isem.at[slot])

    def _wait_idx(c):
        slot = c % 6
        pltpu.make_async_copy(esrc_hbm.at[pl.ds(base + K * c, K)],
                              rsrc.at[slot], isem.at[slot]).wait()
        pltpu.make_async_copy(edst_hbm.at[pl.ds(base + K * c, K)],
                              rdst.at[slot], isem.at[slot]).wait()

    for c in range(4):
        _issue_idx(c)
    for c in range(2):
        _wait_idx(c)
        pltpu.async_copy(h_hbm.at[rsrc.at[c]], gbuf.at[c], gsem.at[c])

    def _chunk_body(c, _):
        b = c % 2
        slot = c % 6
        pltpu.make_async_copy(h_hbm.at[rsrc.at[slot]], gbuf.at[b],
                              gsem.at[b]).wait()

        @pl.when(c >= 2)
        def _():
            pltpu.make_async_copy(sbuf.at[b], u_sh.at[rdst.at[slot]],
                                  ssem.at[b]).wait()

        @pl.when(c + 4 < NCH)
        def _():
            _issue_idx(c + 4)

        for j in range(K // 16):
            sv = rsrc[slot, pl.ds(16 * j, 16)]
            dv = rdst[slot, pl.ds(16 * j, 16)]
            av = plsc.load_gather(asrc_v, [sv >> 7, sv & 127])
            bv = plsc.load_gather(adst_v, [dv >> 7, dv & 127])
            e = av + bv
            e = jnp.where(e >= 0.0, e, e * 0.2)
            ee = jnp.exp(e)
            lim = jnp.full((16,), EPW - K * c - 16 * j, jnp.int32)
            ee = jnp.where(iota16 < lim, ee, 0.0)
            ree[0, pl.ds(16 * j, 16)] = ee
            plsc.addupdate_scatter(s_local, [dv >> 7, dv & 127], ee)

        @plsc.parallel_loop(0, K, unroll=16)
        def _scale_body(e):
            eev = plsc.load_gather(
                ree, [jnp.zeros((16,), jnp.int32),
                      jnp.full((16,), e, jnp.int32)])
            for j in range(D // 16):
                sbuf[b, e, pl.ds(16 * j, 16)] = (
                    gbuf[b, e, pl.ds(16 * j, 16)] * eev)

        @pl.when(c + 2 < NCH)
        def _():
            _wait_idx(c + 2)
            pltpu.async_copy(h_hbm.at[rsrc.at[(c + 2) % 6]], gbuf.at[b],
                             gsem.at[b])

        pltpu.async_copy(sbuf.at[b], u_sh.at[rdst.at[slot]], ssem.at[b],
                         add=True)
        return 0
    lax.fori_loop(0, NCH, _chunk_body, 0)

    for t in (NCH - 2, NCH - 1):
        pltpu.make_async_copy(sbuf.at[t % 2], u_sh.at[rdst.at[t % 6]],
                              ssem.at[t % 2]).wait()

    pltpu.sync_copy(s_local, u_sh.at[iota_v], add=True)

    plsc.subcore_barrier()
    pltpu.sync_copy(u_sh.at[pl.ds(sid * (NPU // 16), NPU // 16)],
                    u_hbm.at[cid, pl.ds(sid * (NPU // 16), NPU // 16)])

    @pl.when(sid < NR // 8)
    def _():
        pltpu.sync_copy(u_sh.at[pl.ds(SROW + sid * 8, 8)],
                        sd_hbm.at[cid, pl.ds(sid * 8, 8)])


_sc_agg = functools.partial(
    pl.kernel,
    out_type=(
        jax.ShapeDtypeStruct((2, NP, D), jnp.float32),
        jax.ShapeDtypeStruct((2, NR, D), jnp.float32),
    ),
    mesh=plsc.VectorSubcoreMesh(core_axis_name="c", subcore_axis_name="s"),
    compiler_params=pltpu.CompilerParams(needs_layout_passes=False),
    scratch_types=[
        pltpu.VMEM((6, K), jnp.int32),
        pltpu.VMEM((6, K), jnp.int32),
        pltpu.VMEM((1, K), jnp.float32),
        pltpu.VMEM((NR, D), jnp.float32),
        pltpu.VMEM((NR, D), jnp.float32),
        pltpu.VMEM((NR, D), jnp.float32),
        pltpu.VMEM((NR,), jnp.int32),
        pltpu.VMEM((2, K, D), jnp.float32),
        pltpu.VMEM((2, K, D), jnp.float32),
        pltpu.SemaphoreType.DMA((2,)),
        pltpu.SemaphoreType.DMA((2,)),
        pltpu.SemaphoreType.DMA((6,)),
        pltpu.VMEM_SHARED((NPU, D), jnp.float32),
    ],
)(_sc_body)


def kernel(x, edge_index, edge_weight, W0, a0s, a0d, b0, W1, a1s, a1d, b1,
           W2, a2s, a2d, b2, Wm1, bm1, Wm2, bm2):
    del edge_weight
    xp = jnp.zeros((NP, D), jnp.float32).at[:N].set(x)
    eidx = edge_index.astype(jnp.int32)
    w2p = jnp.zeros((HID, D), jnp.float32).at[:, :C].set(Wm2)
    b2p = jnp.zeros((D,), jnp.float32).at[:C].set(bm2)
    esrc = jnp.zeros((NW, EPWP), jnp.int32).at[:, :EPW].set(
        eidx[0].reshape(NW, EPW)).reshape(-1)
    edst = jnp.full((NW, EPWP), TRASH, jnp.int32).at[:, :EPW].set(
        eidx[1].reshape(NW, EPW)).reshape(-1)

    h, a_s, a_d = _tc_first(xp, W0, a0s, a0d)
    u, sd = _sc_agg(h, a_s, a_d, esrc, edst)
    h, a_s, a_d = _tc_mid(u, sd, b0, W1, a1s, a1d)
    u, sd = _sc_agg(h, a_s, a_d, esrc, edst)
    h, a_s, a_d = _tc_mid(u, sd, b1, W2, a2s, a2d)
    u, sd = _sc_agg(h, a_s, a_d, esrc, edst)
    y = _tc_head(u, sd, b2, Wm1, bm1, w2p, b2p)
    return y[:N, :C]


# speedup vs baseline: 1.0124x; 1.0030x over previous
import functools

import jax
import jax.numpy as jnp
from jax import lax
from jax.experimental import pallas as pl
from jax.experimental.pallas import tpu as pltpu
from jax.experimental.pallas import tpu_sc as plsc

N = 10000
NP = 10240
D = 128
E = 320000
HID = 256
C = 6

NW = 32
EPW = E // NW
K = 32
NCH = 313
EPWP = NCH * K
TRASH = N + 100
NPU = 10112
SROW = 10016
BM = 1024
NR = NP // D
RPT = NR // 16


def _first_body(x_ref, w_ref, as_ref, ad_ref, h_ref, s_ref, d_ref):
    h = jnp.dot(x_ref[...], w_ref[...], preferred_element_type=jnp.float32)
    h_ref[...] = h
    s_ref[...] = jnp.dot(h, as_ref[...],
                         preferred_element_type=jnp.float32).reshape(
                             BM // D, D)
    d_ref[...] = jnp.dot(h, ad_ref[...],
                         preferred_element_type=jnp.float32).reshape(
                             BM // D, D)


def _tc_first(xp, w, a_s, a_d):
    return pl.pallas_call(
        _first_body,
        grid=(NP // BM,),
        in_specs=[
            pl.BlockSpec((BM, D), lambda i: (i, 0)),
            pl.BlockSpec((D, D), lambda i: (0, 0)),
            pl.BlockSpec((D,), lambda i: (0,)),
            pl.BlockSpec((D,), lambda i: (0,)),
        ],
        out_specs=[
            pl.BlockSpec((BM, D), lambda i: (i, 0)),
            pl.BlockSpec((BM // D, D), lambda i: (i, 0)),
            pl.BlockSpec((BM // D, D), lambda i: (i, 0)),
        ],
        out_shape=[
            jax.ShapeDtypeStruct((NP, D), jnp.float32),
            jax.ShapeDtypeStruct((NR, D), jnp.float32),
            jax.ShapeDtypeStruct((NR, D), jnp.float32),
        ],
    )(xp, w, a_s, a_d)


def _combine(u_ref, sd_ref, b_ref):
    u = u_ref[0] + u_ref[1]
    s = sd_ref[0] + sd_ref[1]
    recip = 1.0 / jnp.where(s == 0.0, 1.0, s)
    eye = (lax.broadcasted_iota(jnp.int32, (D, D), 0)
           == lax.broadcasted_iota(jnp.int32, (D, D), 1))
    rows = []
    for g in range(BM // D):
        dg = jnp.where(eye, recip[g][None, :], 0.0)
        rows.append(jnp.dot(dg, u[g * D:(g + 1) * D],
                            preferred_element_type=jnp.float32))
    return jnp.maximum(jnp.concatenate(rows, axis=0) + b_ref[...], 0.0)


def _mid_body(u_ref, sd_ref, b_ref, w_ref, as_ref, ad_ref,
              h_ref, s_ref, d_ref):
    o = _combine(u_ref, sd_ref, b_ref)
    h = jnp.dot(o, w_ref[...], preferred_element_type=jnp.float32)
    h_ref[...] = h
    s_ref[...] = jnp.dot(h, as_ref[...],
                         preferred_element_type=jnp.float32).reshape(
                             BM // D, D)
    d_ref[...] = jnp.dot(h, ad_ref[...],
                         preferred_element_type=jnp.float32).reshape(
                             BM // D, D)


def _tc_mid(u, sd, b, w, a_s, a_d):
    return pl.pallas_call(
        _mid_body,
        grid=(NP // BM,),
        in_specs=[
            pl.BlockSpec((2, BM, D), lambda i: (0, i, 0)),
            pl.BlockSpec((2, BM // D, D), lambda i: (0, i, 0)),
            pl.BlockSpec((D,), lambda i: (0,)),
            pl.BlockSpec((D, D), lambda i: (0, 0)),
            pl.BlockSpec((D,), lambda i: (0,)),
            pl.BlockSpec((D,), lambda i: (0,)),
        ],
        out_specs=[
            pl.BlockSpec((BM, D), lambda i: (i, 0)),
            pl.BlockSpec((BM // D, D), lambda i: (i, 0)),
            pl.BlockSpec((BM // D, D), lambda i: (i, 0)),
        ],
        out_shape=[
            jax.ShapeDtypeStruct((NP, D), jnp.float32),
            jax.ShapeDtypeStruct((NR, D), jnp.float32),
            jax.ShapeDtypeStruct((NR, D), jnp.float32),
        ],
    )(u, sd, b, w, a_s, a_d)


def _head_body(u_ref, sd_ref, b_ref, w1_ref, b1_ref, w2_ref, b2_ref, y_ref):
    o = _combine(u_ref, sd_ref, b_ref)
    t = jnp.maximum(
        jnp.dot(o, w1_ref[...], preferred_element_type=jnp.float32)
        + b1_ref[...], 0.0)
    y_ref[...] = (jnp.dot(t, w2_ref[...], preferred_element_type=jnp.float32)
                  + b2_ref[...])


def _tc_head(u, sd, b, w1, b1, w2p, b2p):
    return pl.pallas_call(
        _head_body,
        grid=(NP // BM,),
        in_specs=[
            pl.BlockSpec((2, BM, D), lambda i: (0, i, 0)),
            pl.BlockSpec((2, BM // D, D), lambda i: (0, i, 0)),
            pl.BlockSpec((D,), lambda i: (0,)),
            pl.BlockSpec((D, HID), lambda i: (0, 0)),
            pl.BlockSpec((HID,), lambda i: (0,)),
            pl.BlockSpec((HID, D), lambda i: (0, 0)),
            pl.BlockSpec((D,), lambda i: (0,)),
        ],
        out_specs=[pl.BlockSpec((BM, D), lambda i: (i, 0))],
        out_shape=[jax.ShapeDtypeStruct((NP, D), jnp.float32)],
    )(u, sd, b, w1, b1, w2p, b2p)[0]


def _sc_body(h_hbm, asrc_hbm, adst_hbm, esrc_hbm, edst_hbm,
             u_hbm, sd_hbm,
             rsrc, rdst, ree, asrc_v, adst_v, s_local, iota_v,
             gbuf, sbuf, gsem, ssem, isem, u_sh):
    cid = lax.axis_index("c")
    sid = lax.axis_index("s")
    wid = sid * 2 + cid
    base = wid * EPWP

    zvec = jnp.zeros((16,), jnp.float32)
    iota16 = lax.iota(jnp.int32, 16)

    def _zero_body(r, _):
        for j in range(D // 16):
            sbuf[0, r, pl.ds(16 * j, 16)] = zvec
        return 0
    lax.fori_loop(0, K, _zero_body, 0)
    for kk in range(NPU // 16 // K):
        pltpu.sync_copy(sbuf.at[0],
                        u_sh.at[pl.ds(sid * (NPU // 16) + kk * K, K)])
    pltpu.sync_copy(sbuf.at[0, pl.ds(0, (NPU // 16) % K)],
                    u_sh.at[pl.ds(sid * (NPU // 16) + (NPU // 16 // K) * K,
                                  (NPU // 16) % K)])
    plsc.subcore_barrier()

    def _zs_body(r, _):
        for j in range(D // 16):
            s_local[r, pl.ds(16 * j, 16)] = zvec
        return 0
    lax.fori_loop(0, NR, _zs_body, 0)
    for i in range(NR // 16):
        iota_v[pl.ds(16 * i, 16)] = iota16 + (16 * i + SROW)
    pltpu.sync_copy(asrc_hbm, asrc_v)
    pltpu.sync_copy(adst_hbm, adst_v)

    def _issue_idx(c):
        slot = c % 6
        pltpu.async_copy(esrc_hbm.at[pl.ds(base + K * c, K)],
                         rsrc.at[slot], isem.at[slot])
        pltpu.async_copy(edst_hbm.at[pl.ds(base + K * c, K)],
                         rdst.at[slot], isem.at[slot])

    def _wait_idx(c):
        slot = c % 6
        pltpu.make_async_copy(esrc_hbm.at[pl.ds(base + K * c, K)],
                              rsrc.at[slot], isem.at[slot]).wait()
        pltpu.make_async_copy(edst_hbm.at[pl.ds(base + K * c, K)],
                              rdst.at[slot], isem.at[slot]).wait()

    for c in range(4):
        _issue_idx(c)
    for c in range(2):
        _wait_idx(c)
        pltpu.async_copy(h_hbm.at[rsrc.at[c]], gbuf.at[c], gsem.at[c])

    def _chunk_body(c, _):
        b = c % 2
        slot = c % 6
        pltpu.make_async_copy(h_hbm.at[rsrc.at[slot]], gbuf.at[b],
                              gsem.at[b]).wait()

        @pl.when(c >= 2)
        def _():
            pltpu.make_async_copy(sbuf.at[b], u_sh.at[rdst.at[slot]],
                                  ssem.at[b]).wait()

        @pl.when(c + 4 < NCH)
        def _():
            _issue_idx(c + 4)

        for j in range(K // 16):
            sv = rsrc[slot, pl.ds(16 * j, 16)]
            dv = rdst[slot, pl.ds(16 * j, 16)]
            av = plsc.load_gather(asrc_v, [sv >> 7, sv & 127])
            bv = plsc.load_gather(adst_v, [dv >> 7, dv & 127])
            e = av + bv
            e = jnp.where(e >= 0.0, e, e * 0.2)
            ee = jnp.exp(e)
            lim = jnp.full((16,), EPW - K * c - 16 * j, jnp.int32)
            ee = jnp.where(iota16 < lim, ee, 0.0)
            ree[0, pl.ds(16 * j, 16)] = ee
            plsc.addupdate_scatter(s_local, [dv >> 7, dv & 127], ee)

        @plsc.parallel_loop(0, K, unroll=8)
        def _scale_body(e):
            eev = plsc.load_gather(
                ree, [jnp.zeros((16,), jnp.int32),
                      jnp.full((16,), e, jnp.int32)])
            for j in range(D // 16):
                sbuf[b, e, pl.ds(16 * j, 16)] = (
                    gbuf[b, e, pl.ds(16 * j, 16)] * eev)

        @pl.when(c + 2 < NCH)
        def _():
            _wait_idx(c + 2)
            pltpu.async_copy(h_hbm.at[rsrc.at[(c + 2) % 6]], gbuf.at[b],
                             gsem.at[b])

        pltpu.async_copy(sbuf.at[b], u_sh.at[rdst.at[slot]], ssem.at[b],
                         add=True)
        return 0
    lax.fori_loop(0, NCH, _chunk_body, 0)

    for t in (NCH - 2, NCH - 1):
        pltpu.make_async_copy(sbuf.at[t % 2], u_sh.at[rdst.at[t % 6]],
                              ssem.at[t % 2]).wait()

    pltpu.sync_copy(s_local, u_sh.at[iota_v], add=True)

    plsc.subcore_barrier()
    pltpu.sync_copy(u_sh.at[pl.ds(sid * (NPU // 16), NPU // 16)],
                    u_hbm.at[cid, pl.ds(sid * (NPU // 16), NPU // 16)])

    @pl.when(sid < NR // 8)
    def _():
        pltpu.sync_copy(u_sh.at[pl.ds(SROW + sid * 8, 8)],
                        sd_hbm.at[cid, pl.ds(sid * 8, 8)])


_sc_agg = functools.partial(
    pl.kernel,
    out_type=(
        jax.ShapeDtypeStruct((2, NP, D), jnp.float32),
        jax.ShapeDtypeStruct((2, NR, D), jnp.float32),
    ),
    mesh=plsc.VectorSubcoreMesh(core_axis_name="c", subcore_axis_name="s"),
    compiler_params=pltpu.CompilerParams(needs_layout_passes=False),
    scratch_types=[
        pltpu.VMEM((6, K), jnp.int32),
        pltpu.VMEM((6, K), jnp.int32),
        pltpu.VMEM((1, K), jnp.float32),
        pltpu.VMEM((NR, D), jnp.float32),
        pltpu.VMEM((NR, D), jnp.float32),
        pltpu.VMEM((NR, D), jnp.float32),
        pltpu.VMEM((NR,), jnp.int32),
        pltpu.VMEM((2, K, D), jnp.float32),
        pltpu.VMEM((2, K, D), jnp.float32),
        pltpu.SemaphoreType.DMA((2,)),
        pltpu.SemaphoreType.DMA((2,)),
        pltpu.SemaphoreType.DMA((6,)),
        pltpu.VMEM_SHARED((NPU, D), jnp.float32),
    ],
)(_sc_body)


def kernel(x, edge_index, edge_weight, W0, a0s, a0d, b0, W1, a1s, a1d, b1,
           W2, a2s, a2d, b2, Wm1, bm1, Wm2, bm2):
    del edge_weight
    xp = jnp.zeros((NP, D), jnp.float32).at[:N].set(x)
    eidx = edge_index.astype(jnp.int32)
    w2p = jnp.zeros((HID, D), jnp.float32).at[:, :C].set(Wm2)
    b2p = jnp.zeros((D,), jnp.float32).at[:C].set(bm2)
    esrc = jnp.zeros((NW, EPWP), jnp.int32).at[:, :EPW].set(
        eidx[0].reshape(NW, EPW)).reshape(-1)
    edst = jnp.full((NW, EPWP), TRASH, jnp.int32).at[:, :EPW].set(
        eidx[1].reshape(NW, EPW)).reshape(-1)

    h, a_s, a_d = _tc_first(xp, W0, a0s, a0d)
    u, sd = _sc_agg(h, a_s, a_d, esrc, edst)
    h, a_s, a_d = _tc_mid(u, sd, b0, W1, a1s, a1d)
    u, sd = _sc_agg(h, a_s, a_d, esrc, edst)
    h, a_s, a_d = _tc_mid(u, sd, b1, W2, a2s, a2d)
    u, sd = _sc_agg(h, a_s, a_d, esrc, edst)
    y = _tc_head(u, sd, b2, Wm1, bm1, w2p, b2p)
    return y[:N, :C]
